# Initial kernel scaffold; baseline (speedup 1.0000x reference)
import jax
import jax.numpy as jnp
from jax.experimental import pallas as pl


def kernel(x, edge_index, W, b):
    raise NotImplementedError("write your pallas kernel here")


# speedup vs baseline: 20.6988x; 20.6988x over previous
import functools

import jax
import jax.numpy as jnp
from jax import lax
from jax.experimental import pallas as pl
from jax.experimental.pallas import tpu as pltpu
from jax.experimental.pallas import tpu_sc as plsc

N_NODES = 10000
D = 128
E_EDGES = 320000

NP = 10240
NCORES = 2
NSUB = 16
NTILES = NCORES * NSUB
RPT = NP // NSUB
CHUNK = 128
CPT = 79
E_PAD = NTILES * CPT * CHUNK

_MESH = plsc.VectorSubcoreMesh(
    core_axis_name="c", subcore_axis_name="s",
    num_cores=NCORES, num_subcores=NSUB)


@functools.partial(
    pl.kernel,
    out_type=jax.ShapeDtypeStruct((NCORES, NP), jnp.float32),
    mesh=_MESH,
    scratch_types=[
        pltpu.VMEM((CPT, CHUNK), jnp.int32),
        pltpu.VMEM((CHUNK,), jnp.float32),
        pltpu.VMEM_SHARED((NP,), jnp.float32),
    ],
)
def _deg_kernel(dst_hbm, ones_hbm, zdeg_hbm, out_hbm, dst_v, ones_v, deg_sh):
    cid = lax.axis_index("c")
    sid = lax.axis_index("s")
    gid = cid * NSUB + sid
    pltpu.sync_copy(zdeg_hbm, deg_sh.at[pl.ds(sid * RPT, RPT)])
    pltpu.sync_copy(dst_hbm.at[gid], dst_v)
    pltpu.sync_copy(ones_hbm, ones_v)
    plsc.subcore_barrier()

    def body(j, carry):
        pltpu.sync_copy(ones_v, deg_sh.at[dst_v.at[j]], add=True)
        return carry

    lax.fori_loop(0, CPT, body, 0)
    plsc.subcore_barrier()
    pltpu.sync_copy(deg_sh.at[pl.ds(sid * RPT, RPT)],
                    out_hbm.at[cid, pl.ds(sid * RPT, RPT)])


@functools.partial(
    pl.kernel,
    out_type=jax.ShapeDtypeStruct((NCORES, NP, D), jnp.float32),
    mesh=_MESH,
    scratch_types=[
        pltpu.VMEM((CPT, CHUNK), jnp.int32),
        pltpu.VMEM((CPT, CHUNK), jnp.int32),
        pltpu.VMEM((CHUNK, D), jnp.float32),
        pltpu.VMEM_SHARED((NP, D), jnp.float32),
        pltpu.SemaphoreType.DMA,
    ],
)
def _acc_kernel(src_hbm, dst_hbm, g_hbm, zrow_hbm, out_hbm,
                src_v, dst_v, gbuf, acc_sh, sem):
    cid = lax.axis_index("c")
    sid = lax.axis_index("s")
    gid = cid * NSUB + sid
    pltpu.sync_copy(zrow_hbm, acc_sh.at[pl.ds(sid * RPT, RPT)])
    pltpu.sync_copy(src_hbm.at[gid], src_v)
    pltpu.sync_copy(dst_hbm.at[gid], dst_v)
    plsc.subcore_barrier()

    def body(j, carry):
        pltpu.async_copy(g_hbm.at[src_v.at[j]], gbuf, sem).wait()
        pltpu.sync_copy(gbuf, acc_sh.at[dst_v.at[j]], add=True)
        return carry

    lax.fori_loop(0, CPT, body, 0)
    plsc.subcore_barrier()
    pltpu.sync_copy(acc_sh.at[pl.ds(sid * RPT, RPT)],
                    out_hbm.at[cid, pl.ds(sid * RPT, RPT)])


_BLK_G = 1280


def _g_body(x_ref, w_ref, deg_ref, g_ref):
    h = jnp.dot(x_ref[...], w_ref[...], preferred_element_type=jnp.float32)
    deg = deg_ref[0] + deg_ref[1] + 1.0
    dinv = lax.rsqrt(jnp.maximum(deg, 1e-12))
    g_ref[...] = h * dinv


def _g_call(xp, W, deg3):
    return pl.pallas_call(
        _g_body,
        grid=(NP // _BLK_G,),
        in_specs=[
            pl.BlockSpec((_BLK_G, D), lambda i: (i, 0)),
            pl.BlockSpec((D, D), lambda i: (0, 0)),
            pl.BlockSpec((NCORES, _BLK_G, 1), lambda i: (0, i, 0)),
        ],
        out_specs=pl.BlockSpec((_BLK_G, D), lambda i: (i, 0)),
        out_shape=jax.ShapeDtypeStruct((NP, D), jnp.float32),
    )(xp, W, deg3)


_BLK_O = 1000


def _out_body(acc_ref, g_ref, deg_ref, b_ref, o_ref):
    a = acc_ref[0] + acc_ref[1] + g_ref[...]
    deg = deg_ref[0] + deg_ref[1] + 1.0
    dinv = lax.rsqrt(jnp.maximum(deg, 1e-12))
    t = a * dinv + b_ref[...]
    o_ref[...] = 0.5 * t * (1.0 + lax.erf(t * 0.7071067811865476))


def _out_call(acc3, g, deg3, b2):
    return pl.pallas_call(
        _out_body,
        grid=(N_NODES // _BLK_O,),
        in_specs=[
            pl.BlockSpec((NCORES, _BLK_O, D), lambda i: (0, i, 0)),
            pl.BlockSpec((_BLK_O, D), lambda i: (i, 0)),
            pl.BlockSpec((NCORES, _BLK_O, 1), lambda i: (0, i, 0)),
            pl.BlockSpec((1, D), lambda i: (0, 0)),
        ],
        out_specs=pl.BlockSpec((_BLK_O, D), lambda i: (i, 0)),
        out_shape=jax.ShapeDtypeStruct((N_NODES, D), jnp.float32),
    )(acc3, g, deg3, b2)


def kernel(x, edge_index, W, b):
    src = edge_index[0]
    dst = edge_index[1]
    pad = jnp.full((E_PAD - E_EDGES,), N_NODES, dtype=jnp.int32)
    src_r = jnp.concatenate([src, pad]).reshape(NTILES, CPT, CHUNK)
    dst_r = jnp.concatenate([dst, pad]).reshape(NTILES, CPT, CHUNK)
    xp = jnp.zeros((NP, D), jnp.float32).at[:N_NODES].set(x)

    ones1d = jnp.ones((CHUNK,), jnp.float32)
    zdeg = jnp.zeros((RPT,), jnp.float32)
    zrow = jnp.zeros((RPT, D), jnp.float32)

    deg2 = _deg_kernel(dst_r, ones1d, zdeg)
    deg3 = deg2.reshape(NCORES, NP, 1)
    g = _g_call(xp, W, deg3)
    acc3 = _acc_kernel(src_r, dst_r, g, zrow)
    return _out_call(acc3, g, deg3, b[None, :])
